# baseline (speedup 1.0000x reference)
import functools

import jax
import jax.numpy as jnp
from jax import lax
from jax.experimental import pallas as pl
from jax.experimental.pallas import tpu as pltpu
from jax.experimental.pallas import tpu_sc as plsc

N = 8192
K = 8192
D = 32
TN = 256
TKC = 128
NT = N // TN
KC = K // TKC
E2CHUNK = 1024


def _e2_body(e_ref, e2b_ref):
    ej = e_ref[...]
    e2j = jnp.sum(ej * ej, axis=1, keepdims=True)
    e2b_ref[...] = jnp.broadcast_to(e2j, (E2CHUNK, TN))


def _dist_body(e_ref, ftm2_ref, e2_ref, minv_ref, mini_ref):
    ftm2 = ftm2_ref[...]

    def step(j, carry):
        mv, mc = carry
        e = e_ref[pl.ds(j * TKC, TKC), :]
        prod = lax.dot_general(e, ftm2, (((1,), (0,)), ((), ())),
                               preferred_element_type=jnp.float32)
        s = prod + e2_ref[pl.ds(j * TKC, TKC), :]
        return jnp.minimum(s, mv), jnp.where(s < mv, j, mc)

    mv0 = jnp.full((TKC, TN), jnp.inf, dtype=jnp.float32)
    mc0 = jnp.zeros((TKC, TN), dtype=jnp.int32)
    mv, mc = lax.fori_loop(0, KC, step, (mv0, mc0), unroll=64)

    idx = mc * TKC + lax.broadcasted_iota(jnp.int32, (TKC, TN), 0)
    m = jnp.min(mv, axis=0, keepdims=True)
    targ = jnp.min(jnp.where(mv == m, idx, K), axis=0, keepdims=True)
    x2 = 0.25 * jnp.sum(ftm2 * ftm2, axis=0, keepdims=True)
    minv_ref[0] = jnp.maximum(m + x2, 0.0)
    mini_ref[0] = targ


def _dist_argmin(e, ftm2):
    e2b = pl.pallas_call(
        _e2_body,
        grid=(K // E2CHUNK,),
        in_specs=[pl.BlockSpec((E2CHUNK, D), lambda n: (n, 0))],
        out_specs=pl.BlockSpec((E2CHUNK, TN), lambda n: (n, 0)),
        out_shape=jax.ShapeDtypeStruct((K, TN), jnp.float32),
        compiler_params=pltpu.CompilerParams(
            dimension_semantics=("parallel",)),
    )(e)
    minv, mini = pl.pallas_call(
        _dist_body,
        grid=(NT,),
        in_specs=[
            pl.BlockSpec((K, D), lambda n: (0, 0)),
            pl.BlockSpec((D, TN), lambda n: (0, n)),
            pl.BlockSpec((K, TN), lambda n: (0, 0)),
        ],
        out_specs=[
            pl.BlockSpec((1, 1, TN), lambda n: (n, 0, 0)),
            pl.BlockSpec((1, 1, TN), lambda n: (n, 0, 0)),
        ],
        out_shape=[
            jax.ShapeDtypeStruct((NT, 1, TN), jnp.float32),
            jax.ShapeDtypeStruct((NT, 1, TN), jnp.int32),
        ],
        compiler_params=pltpu.CompilerParams(
            dimension_semantics=("parallel",)),
    )(e, ftm2, e2b)
    return minv.reshape(N), mini.reshape(N)


_SC_INFO = plsc.get_sparse_core_info()
_NW = _SC_INFO.num_cores * _SC_INFO.num_subcores
_BPW = N // _NW


@functools.partial(
    pl.kernel,
    mesh=plsc.VectorSubcoreMesh(core_axis_name="c", subcore_axis_name="s"),
    out_type=jax.ShapeDtypeStruct((N, D), jnp.float32),
    scratch_types=[
        pltpu.VMEM((_BPW,), jnp.int32),
        pltpu.VMEM((_BPW, D), jnp.float32),
        pltpu.SemaphoreType.DMA,
    ],
    compiler_params=pltpu.CompilerParams(use_tc_tiling_on_sc=False),
)
def _sc_gather(table_hbm, idx_hbm, out_hbm, idx_v, rows_v, sem):
    wid = lax.axis_index("s") * _SC_INFO.num_cores + lax.axis_index("c")
    base = wid * _BPW
    pltpu.sync_copy(idx_hbm.at[pl.ds(base, _BPW)], idx_v)
    pltpu.async_copy(table_hbm.at[idx_v], rows_v, sem).wait()
    pltpu.sync_copy(rows_v, out_hbm.at[pl.ds(base, _BPW)])


def kernel(x, embed):
    x = x.astype(jnp.float32)
    f = x.reshape(N, D)
    e = embed.reshape(K, D).astype(jnp.float32)
    ftm2 = -2.0 * f.T
    minv, mini = _dist_argmin(e, ftm2)
    quantize = _sc_gather(e, mini)
    return quantize, mini.reshape(1, N), minv.reshape(1, N)


# speedup vs baseline: 1.1381x; 1.1381x over previous
import functools

import jax
import jax.numpy as jnp
from jax import lax
from jax.experimental import pallas as pl
from jax.experimental.pallas import tpu as pltpu
from jax.experimental.pallas import tpu_sc as plsc

N = 8192
K = 8192
D = 32
TN = 256
TKC = 128
NT = N // TN
KC = K // TKC
E2CHUNK = 1024


def _dist_body(e_ref, ftm2_ref, minv_ref, mini_ref, e2_ref):
    n = pl.program_id(0)

    @pl.when(n == 0)
    def _():
        for jj in range(K // E2CHUNK):
            ej = e_ref[jj * E2CHUNK:(jj + 1) * E2CHUNK, :]
            e2j = jnp.sum(ej * ej, axis=1, keepdims=True)
            e2_ref[jj * E2CHUNK:(jj + 1) * E2CHUNK, :] = jnp.broadcast_to(
                e2j, (E2CHUNK, TN))

    ftm2 = ftm2_ref[...]

    def step(j, carry):
        mv, mc = carry
        e = e_ref[pl.ds(j * TKC, TKC), :]
        prod = lax.dot_general(e, ftm2, (((1,), (0,)), ((), ())),
                               preferred_element_type=jnp.float32)
        s = prod + e2_ref[pl.ds(j * TKC, TKC), :]
        return jnp.minimum(s, mv), jnp.where(s < mv, j, mc)

    mv0 = jnp.full((TKC, TN), jnp.inf, dtype=jnp.float32)
    mc0 = jnp.zeros((TKC, TN), dtype=jnp.int32)
    mv, mc = lax.fori_loop(0, KC, step, (mv0, mc0), unroll=64)

    idx = mc * TKC + lax.broadcasted_iota(jnp.int32, (TKC, TN), 0)
    m = jnp.min(mv, axis=0, keepdims=True)
    targ = jnp.min(jnp.where(mv == m, idx, K), axis=0, keepdims=True)
    x2 = 0.25 * jnp.sum(ftm2 * ftm2, axis=0, keepdims=True)
    minv_ref[0] = jnp.maximum(m + x2, 0.0)
    mini_ref[0] = targ


def _dist_argmin(e, ftm2):
    minv, mini = pl.pallas_call(
        _dist_body,
        grid=(NT,),
        in_specs=[
            pl.BlockSpec((K, D), lambda n: (0, 0)),
            pl.BlockSpec((D, TN), lambda n: (0, n)),
        ],
        out_specs=[
            pl.BlockSpec((1, 1, TN), lambda n: (n, 0, 0)),
            pl.BlockSpec((1, 1, TN), lambda n: (n, 0, 0)),
        ],
        out_shape=[
            jax.ShapeDtypeStruct((NT, 1, TN), jnp.float32),
            jax.ShapeDtypeStruct((NT, 1, TN), jnp.int32),
        ],
        scratch_shapes=[pltpu.VMEM((K, TN), jnp.float32)],
    )(e, ftm2)
    return minv.reshape(N), mini.reshape(N)


_SC_INFO = plsc.get_sparse_core_info()
_NW = _SC_INFO.num_cores * _SC_INFO.num_subcores
_BPW = N // _NW


@functools.partial(
    pl.kernel,
    mesh=plsc.VectorSubcoreMesh(core_axis_name="c", subcore_axis_name="s"),
    out_type=jax.ShapeDtypeStruct((N, D), jnp.float32),
    scratch_types=[
        pltpu.VMEM((_BPW,), jnp.int32),
        pltpu.VMEM((_BPW, D), jnp.float32),
        pltpu.SemaphoreType.DMA,
    ],
    compiler_params=pltpu.CompilerParams(use_tc_tiling_on_sc=False),
)
def _sc_gather(table_hbm, idx_hbm, out_hbm, idx_v, rows_v, sem):
    wid = lax.axis_index("s") * _SC_INFO.num_cores + lax.axis_index("c")
    base = wid * _BPW
    pltpu.sync_copy(idx_hbm.at[pl.ds(base, _BPW)], idx_v)
    pltpu.async_copy(table_hbm.at[idx_v], rows_v, sem).wait()
    pltpu.sync_copy(rows_v, out_hbm.at[pl.ds(base, _BPW)])


def kernel(x, embed):
    x = x.astype(jnp.float32)
    f = x.reshape(N, D)
    e = embed.reshape(K, D).astype(jnp.float32)
    ftm2 = -2.0 * f.T
    minv, mini = _dist_argmin(e, ftm2)
    quantize = _sc_gather(e, mini)
    return quantize, mini.reshape(1, N), minv.reshape(1, N)
